# Initial kernel scaffold; baseline (speedup 1.0000x reference)
import jax
import jax.numpy as jnp
from jax.experimental import pallas as pl


def kernel(feature, edge_index, W1, b1, W2, b2):
    raise NotImplementedError("write your pallas kernel here")


# speedup vs baseline: 5.2923x; 5.2923x over previous
import functools

import jax
import jax.numpy as jnp
from jax import lax
from jax.experimental import pallas as pl
from jax.experimental.pallas import tpu as pltpu
from jax.experimental.pallas import tpu_sc as plsc

N_NODES = 10000
N_EDGES = 320000
D = 128
NC = 2
NT = 16
PER_TILE = N_EDGES // (NC * NT)
CH = 80
NCH = PER_TILE // CH
NB = 5
BANDR = 2560
NBAND = 4
TRASH = 64
RPT = BANDR // NT
LANES = 16


def _sc_segment_sum(feature, src5, dst5):
    mesh = plsc.VectorSubcoreMesh(core_axis_name="c", subcore_axis_name="s")

    @functools.partial(
        pl.kernel,
        mesh=mesh,
        out_type=jax.ShapeDtypeStruct((NC, NBAND, NT, RPT, D), jnp.float32),
        scratch_types=[
            pltpu.VMEM((NCH, CH), jnp.int32),
            pltpu.VMEM((NCH, CH), jnp.int32),
            pltpu.VMEM((CH,), jnp.int32),
            pltpu.VMEM((NB, CH, D), jnp.float32),
            pltpu.VMEM((RPT, D), jnp.float32),
            pltpu.VMEM_SHARED((BANDR + TRASH, D), jnp.float32),
        ] + [pltpu.SemaphoreType.DMA] * NB,
    )
    def body(feat_hbm, src_hbm, dst_hbm, out_hbm, src_v, dst_v, lidx, rbufs,
             zbuf, agg_sh, *gsems):
        c = lax.axis_index("c")
        s = lax.axis_index("s")

        pltpu.sync_copy(src_hbm.at[c, s], src_v)
        pltpu.sync_copy(dst_hbm.at[c, s], dst_v)

        def zrow(i, carry):
            for j in range(D // LANES):
                zbuf[i, pl.ds(j * LANES, LANES)] = jnp.zeros((LANES,), jnp.float32)
            return carry
        lax.fori_loop(0, RPT, zrow, 0)

        for b in range(NBAND):
            base = b * BANDR

            pltpu.sync_copy(zbuf, agg_sh.at[pl.ds(s * RPT, RPT), :])
            plsc.subcore_barrier()

            for k in range(NB):
                pltpu.async_copy(feat_hbm.at[src_v.at[k]], rbufs.at[k], gsems[k])

            def group(g, carry):
                for k in range(NB):
                    j = g * NB + k
                    pltpu.make_async_copy(
                        feat_hbm.at[pl.ds(0, CH), :], rbufs.at[k], gsems[k]
                    ).wait()
                    for v in range(CH // LANES):
                        dvec = dst_v[j, pl.ds(v * LANES, LANES)]
                        loc = dvec - base
                        msk = (loc >= 0) & (loc < BANDR)
                        spread = BANDR + (dvec & (TRASH - 1))
                        lidx[pl.ds(v * LANES, LANES)] = jnp.where(msk, loc, spread)
                    pltpu.sync_copy(rbufs.at[k], agg_sh.at[lidx], add=True)
                    nxt = j + NB

                    @pl.when(nxt < NCH)
                    def _():
                        pltpu.async_copy(
                            feat_hbm.at[src_v.at[nxt]], rbufs.at[k], gsems[k]
                        )
                return carry

            lax.fori_loop(0, NCH // NB, group, 0)

            plsc.subcore_barrier()

            pltpu.sync_copy(agg_sh.at[pl.ds(s * RPT, RPT), :], out_hbm.at[c, b, s])
            plsc.subcore_barrier()

    return body(feature, src5, dst5)


def _tc_linear(aggT, W2, b2):
    R = 2000
    grid = (N_NODES // R,)

    def mm_body(a_ref, w_ref, b_ref, o_ref):
        a = a_ref[0] + a_ref[1]
        acc = jnp.dot(a, w_ref[...], preferred_element_type=jnp.float32)
        o_ref[...] = acc + b_ref[...]

    return pl.pallas_call(
        mm_body,
        grid=grid,
        in_specs=[
            pl.BlockSpec((2, R, D), lambda i: (0, i, 0)),
            pl.BlockSpec((D, D), lambda i: (0, 0)),
            pl.BlockSpec((1, D), lambda i: (0, 0)),
        ],
        out_specs=pl.BlockSpec((R, D), lambda i: (i, 0)),
        out_shape=jax.ShapeDtypeStruct((N_NODES, D), jnp.float32),
    )(aggT, W2, b2.reshape(1, D))


def kernel(feature, edge_index, W1, b1, W2, b2):
    src5 = edge_index[0].reshape(NC, NT, NCH, CH)
    dst5 = edge_index[1].reshape(NC, NT, NCH, CH)
    aggT = _sc_segment_sum(feature, src5, dst5).reshape(NC, NBAND * BANDR, D)
    return _tc_linear(aggT, W2, b2)


# speedup vs baseline: 6.7375x; 1.2731x over previous
import functools

import jax
import jax.numpy as jnp
from jax import lax
from jax.experimental import pallas as pl
from jax.experimental.pallas import tpu as pltpu
from jax.experimental.pallas import tpu_sc as plsc

N_NODES = 10000
N_EDGES = 320000
D = 128
NC = 2
NT = 16
PER_TILE = N_EDGES // (NC * NT)
CH = 80
NCH = PER_TILE // CH
NB = 5
BANDR = 3712
NBAND = 3
TRASH = 64
RPT = BANDR // NT
ZR = 116
LANES = 16


def _sc_segment_sum(feature, sd6):
    mesh = plsc.VectorSubcoreMesh(core_axis_name="c", subcore_axis_name="s")

    @functools.partial(
        pl.kernel,
        mesh=mesh,
        out_type=jax.ShapeDtypeStruct(
            (NC, NBAND, NT, RPT // ZR, ZR, D), jnp.float32),
        scratch_types=[
            pltpu.VMEM((2, NCH, CH), jnp.int32),
            pltpu.VMEM((CH,), jnp.int32),
            pltpu.VMEM((NB, CH, D), jnp.float32),
            pltpu.VMEM((ZR, D), jnp.float32),
            pltpu.VMEM_SHARED((BANDR + TRASH, D), jnp.float32),
            pltpu.SemaphoreType.DMA,
        ],
    )
    def body(feat_hbm, sd_hbm, out_hbm, sd_v, lidx, rbufs, zbuf, agg_sh, gsem):
        c = lax.axis_index("c")
        s = lax.axis_index("s")

        pltpu.sync_copy(sd_hbm.at[c, s], sd_v)

        def zrow(i, carry):
            for j in range(D // LANES):
                zbuf[i, pl.ds(j * LANES, LANES)] = jnp.zeros((LANES,), jnp.float32)
            return carry
        lax.fori_loop(0, ZR, zrow, 0)

        def band(b, carry):
            base = b * BANDR

            def zcopy(q, qcarry):
                pltpu.sync_copy(zbuf, agg_sh.at[pl.ds(s * RPT + q * ZR, ZR), :])
                return qcarry
            lax.fori_loop(0, RPT // ZR, zcopy, 0)

            plsc.subcore_barrier()

            def prolog(k, kcarry):
                pltpu.async_copy(feat_hbm.at[sd_v.at[0, k]], rbufs.at[k], gsem)
                return kcarry
            lax.fori_loop(0, NB, prolog, 0)

            def step(j, jcarry):
                slot = lax.rem(j, jnp.int32(NB))
                pltpu.make_async_copy(
                    feat_hbm.at[pl.ds(0, CH), :], rbufs.at[slot], gsem
                ).wait()
                for v in range(CH // LANES):
                    dvec = sd_v[1, j, pl.ds(v * LANES, LANES)]
                    loc = dvec - base
                    msk = (loc >= 0) & (loc < BANDR)
                    spread = BANDR + (dvec & (TRASH - 1))
                    lidx[pl.ds(v * LANES, LANES)] = jnp.where(msk, loc, spread)
                pltpu.sync_copy(rbufs.at[slot], agg_sh.at[lidx], add=True)
                nxt = j + NB

                @pl.when(nxt < NCH)
                def _():
                    pltpu.async_copy(feat_hbm.at[sd_v.at[0, nxt]],
                                     rbufs.at[slot], gsem)
                return jcarry

            lax.fori_loop(0, NCH, step, 0)

            plsc.subcore_barrier()

            def wb(q, qcarry):
                pltpu.sync_copy(agg_sh.at[pl.ds(s * RPT + q * ZR, ZR), :],
                                out_hbm.at[c, b, s, q])
                return qcarry
            lax.fori_loop(0, RPT // ZR, wb, 0)

            plsc.subcore_barrier()
            return carry

        lax.fori_loop(0, NBAND, band, 0)

    return body(feature, sd6)


def _tc_linear(aggT, W2, b2):
    R = 2000
    grid = (N_NODES // R,)

    def mm_body(a_ref, w_ref, b_ref, o_ref):
        a = a_ref[0] + a_ref[1]
        acc = jnp.dot(a, w_ref[...], preferred_element_type=jnp.float32)
        o_ref[...] = acc + b_ref[...]

    return pl.pallas_call(
        mm_body,
        grid=grid,
        in_specs=[
            pl.BlockSpec((2, R, D), lambda i: (0, i, 0)),
            pl.BlockSpec((D, D), lambda i: (0, 0)),
            pl.BlockSpec((1, D), lambda i: (0, 0)),
        ],
        out_specs=pl.BlockSpec((R, D), lambda i: (i, 0)),
        out_shape=jax.ShapeDtypeStruct((N_NODES, D), jnp.float32),
    )(aggT, W2, b2.reshape(1, D))


def kernel(feature, edge_index, W1, b1, W2, b2):
    sd6 = edge_index.reshape(2, NC, NT, NCH, CH).transpose(1, 2, 0, 3, 4)
    aggT = _sc_segment_sum(feature, sd6).reshape(NC, NBAND * BANDR, D)
    return _tc_linear(aggT, W2, b2)


# speedup vs baseline: 6.7383x; 1.0001x over previous
import functools

import jax
import jax.numpy as jnp
from jax import lax
from jax.experimental import pallas as pl
from jax.experimental.pallas import tpu as pltpu
from jax.experimental.pallas import tpu_sc as plsc

N_NODES = 10000
N_EDGES = 320000
D = 128
NC = 2
NT = 16
PER_TILE = N_EDGES // (NC * NT)
CH = 80
NCH = PER_TILE // CH
NB = 5
BANDR = 3712
NBAND = 3
TRASH = 128
RPT = BANDR // NT
ZR = 116
LANES = 16


def _sc_segment_sum(feature, sd6):
    mesh = plsc.VectorSubcoreMesh(core_axis_name="c", subcore_axis_name="s")

    @functools.partial(
        pl.kernel,
        mesh=mesh,
        out_type=jax.ShapeDtypeStruct(
            (NC, NBAND, NT, RPT // ZR, ZR, D), jnp.float32),
        scratch_types=[
            pltpu.VMEM((2, NCH, CH), jnp.int32),
            pltpu.VMEM((CH,), jnp.int32),
            pltpu.VMEM((NB, CH, D), jnp.float32),
            pltpu.VMEM((ZR, D), jnp.float32),
            pltpu.VMEM_SHARED((BANDR + TRASH, D), jnp.float32),
            pltpu.SemaphoreType.DMA,
        ],
    )
    def body(feat_hbm, sd_hbm, out_hbm, sd_v, lidx, rbufs, zbuf, agg_sh, gsem):
        c = lax.axis_index("c")
        s = lax.axis_index("s")

        pltpu.sync_copy(sd_hbm.at[c, s], sd_v)

        def zrow(i, carry):
            for j in range(D // LANES):
                zbuf[i, pl.ds(j * LANES, LANES)] = jnp.zeros((LANES,), jnp.float32)
            return carry
        lax.fori_loop(0, ZR, zrow, 0)

        def band(b, carry):
            base = b * BANDR

            def zcopy(q, qcarry):
                pltpu.sync_copy(zbuf, agg_sh.at[pl.ds(s * RPT + q * ZR, ZR), :])
                return qcarry
            lax.fori_loop(0, RPT // ZR, zcopy, 0)

            plsc.subcore_barrier()

            def prolog(k, kcarry):
                pltpu.async_copy(feat_hbm.at[sd_v.at[0, k]], rbufs.at[k], gsem)
                return kcarry
            lax.fori_loop(0, NB, prolog, 0)

            def step(j, jcarry):
                slot = lax.rem(j, jnp.int32(NB))
                pltpu.make_async_copy(
                    feat_hbm.at[pl.ds(0, CH), :], rbufs.at[slot], gsem
                ).wait()
                for v in range(CH // LANES):
                    dvec = sd_v[1, j, pl.ds(v * LANES, LANES)]
                    loc = dvec - base
                    msk = (loc >= 0) & (loc < BANDR)
                    spread = BANDR + (dvec & (TRASH - 1))
                    lidx[pl.ds(v * LANES, LANES)] = jnp.where(msk, loc, spread)
                pltpu.sync_copy(rbufs.at[slot], agg_sh.at[lidx], add=True)
                nxt = j + NB

                @pl.when(nxt < NCH)
                def _():
                    pltpu.async_copy(feat_hbm.at[sd_v.at[0, nxt]],
                                     rbufs.at[slot], gsem)
                return jcarry

            lax.fori_loop(0, NCH, step, 0)

            plsc.subcore_barrier()

            def wb(q, qcarry):
                pltpu.sync_copy(agg_sh.at[pl.ds(s * RPT + q * ZR, ZR), :],
                                out_hbm.at[c, b, s, q])
                return qcarry
            lax.fori_loop(0, RPT // ZR, wb, 0)

            plsc.subcore_barrier()
            return carry

        lax.fori_loop(0, NBAND, band, 0)

    return body(feature, sd6)


def _tc_linear(aggT, W2, b2):
    R = 2000
    grid = (N_NODES // R,)

    def mm_body(a_ref, w_ref, b_ref, o_ref):
        a = a_ref[0] + a_ref[1]
        acc = jnp.dot(a, w_ref[...], preferred_element_type=jnp.float32)
        o_ref[...] = acc + b_ref[...]

    return pl.pallas_call(
        mm_body,
        grid=grid,
        in_specs=[
            pl.BlockSpec((2, R, D), lambda i: (0, i, 0)),
            pl.BlockSpec((D, D), lambda i: (0, 0)),
            pl.BlockSpec((1, D), lambda i: (0, 0)),
        ],
        out_specs=pl.BlockSpec((R, D), lambda i: (i, 0)),
        out_shape=jax.ShapeDtypeStruct((N_NODES, D), jnp.float32),
    )(aggT, W2, b2.reshape(1, D))


def kernel(feature, edge_index, W1, b1, W2, b2):
    sd6 = edge_index.reshape(2, NC, NT, NCH, CH).transpose(1, 2, 0, 3, 4)
    aggT = _sc_segment_sum(feature, sd6).reshape(NC, NBAND * BANDR, D)
    return _tc_linear(aggT, W2, b2)
